# Initial kernel scaffold; baseline (speedup 1.0000x reference)
import jax
import jax.numpy as jnp
from jax.experimental import pallas as pl


def kernel(z, edge_index):
    raise NotImplementedError("write your pallas kernel here")


# speedup vs baseline: 4.4060x; 4.4060x over previous
import functools

import jax
import jax.numpy as jnp
from jax import lax
from jax.experimental import pallas as pl
from jax.experimental.pallas import tpu as pltpu
from jax.experimental.pallas import tpu_sc as plsc

N_NODES = 10000
D = 128
B = 320000

NC = 2
NS = 16
L = 16

F_PER = D // NS
B_PER_CORE = B // NC
E = 6400
N_CHUNK = B_PER_CORE // E
EG = E // L
SLICE = E // NS


def _sc_body(zt_hbm, src_hbm, dst_hbm, out_hbm,
             zslice_v, src_v, dst_v, partial_v, red_v, res_v, stage_sh):
    c = lax.axis_index("c")
    s = lax.axis_index("s")

    pltpu.sync_copy(zt_hbm.at[pl.ds(s * F_PER * N_NODES, F_PER * N_NODES)],
                    zslice_v)

    core_base = c * B_PER_CORE

    def chunk_body(k, carry):
        off = core_base + k * E
        pltpu.sync_copy(src_hbm.at[pl.ds(off, E)], src_v)
        pltpu.sync_copy(dst_hbm.at[pl.ds(off, E)], dst_v)

        def grp(g, carry2):
            sv = src_v[pl.ds(g * L, L)]
            dv = dst_v[pl.ds(g * L, L)]
            acc = jnp.zeros((L,), jnp.float32)
            for f in range(F_PER):
                a = plsc.load_gather(zslice_v, [sv + (f * N_NODES)])
                b = plsc.load_gather(zslice_v, [dv + (f * N_NODES)])
                acc = acc + a * b
            partial_v[pl.ds(g * L, L)] = acc
            return carry2

        lax.fori_loop(0, EG, grp, 0)

        pltpu.sync_copy(partial_v, stage_sh.at[pl.ds(s * E, E)])
        plsc.subcore_barrier()
        for t in range(NS):
            pltpu.sync_copy(stage_sh.at[pl.ds(t * E + s * SLICE, SLICE)],
                            red_v.at[pl.ds(t * SLICE, SLICE)])

        def red(g, carry2):
            tot = jnp.zeros((L,), jnp.float32)
            for t in range(NS):
                tot = tot + red_v[pl.ds(t * SLICE + g * L, L)]
            y = 1.0 / (1.0 + jnp.exp(-tot))
            res_v[pl.ds(g * L, L)] = y
            return carry2

        lax.fori_loop(0, SLICE // L, red, 0)
        pltpu.sync_copy(res_v, out_hbm.at[pl.ds(off + s * SLICE, SLICE)])
        plsc.subcore_barrier()
        return carry

    lax.fori_loop(0, N_CHUNK, chunk_body, 0)


@jax.jit
def _predict(zt, src, dst):
    mesh = plsc.VectorSubcoreMesh(core_axis_name="c", subcore_axis_name="s")
    return pl.kernel(
        _sc_body,
        out_type=jax.ShapeDtypeStruct((B,), jnp.float32),
        mesh=mesh,
        compiler_params=pltpu.CompilerParams(needs_layout_passes=False),
        scratch_types=[
            pltpu.VMEM((F_PER * N_NODES,), jnp.float32),
            pltpu.VMEM((E,), jnp.int32),
            pltpu.VMEM((E,), jnp.int32),
            pltpu.VMEM((E,), jnp.float32),
            pltpu.VMEM((NS * SLICE,), jnp.float32),
            pltpu.VMEM((SLICE,), jnp.float32),
            pltpu.VMEM_SHARED((NS * E,), jnp.float32),
        ],
    )(zt, src, dst)


def kernel(z, edge_index):
    zt = z.T.reshape(-1)
    src = edge_index[0].astype(jnp.int32)
    dst = edge_index[1].astype(jnp.int32)
    return _predict(zt, src, dst)
